# baseline (speedup 1.0000x reference)
import functools

import jax
import jax.numpy as jnp
from jax import lax
from jax.experimental import pallas as pl
from jax.experimental.pallas import tpu as pltpu
from jax.experimental.pallas import tpu_sc as plsc


def _mlp_t_body(tab_ref, w1_ref, w2_ref, out_ref):
    h = jnp.maximum(
        jnp.dot(tab_ref[...], w1_ref[...], preferred_element_type=jnp.float32),
        0.0,
    )
    t = lax.dot_general(
        w2_ref[...], h, (((0,), (1,)), ((), ())),
        preferred_element_type=jnp.float32,
    )
    out_ref[...] = jnp.concatenate(
        [t, jnp.zeros((t.shape[0], out_ref.shape[1] - t.shape[1]), jnp.float32)],
        axis=1,
    )


def _compute_logits_t(table, W1, W2, Vpad):
    V = table.shape[0]
    F = W2.shape[1]
    return pl.pallas_call(
        _mlp_t_body,
        out_shape=jax.ShapeDtypeStruct((F, Vpad), jnp.float32),
    )(table, W1, W2)


@functools.lru_cache(maxsize=None)
def _make_sc_tgather(L, F, BB, Vpad):
    info = plsc.get_sparse_core_info()
    nc, ns = info.num_cores, info.num_subcores
    nw = nc * ns
    NWIN = BB // 128
    NG = nw // NWIN
    CH = 64
    FG = 256
    NCH = FG // CH
    assert Vpad == 1024 and BB % 128 == 0 and NG * FG >= F and F >= FG

    mesh = plsc.VectorSubcoreMesh(core_axis_name="c", subcore_axis_name="s")

    @functools.partial(
        pl.kernel,
        mesh=mesh,
        out_type=jax.ShapeDtypeStruct((L, F, BB), jnp.float32),
        scratch_types=[
            pltpu.VMEM((L, 128), jnp.int32),
            pltpu.VMEM((CH * Vpad,), jnp.float32),
            pltpu.VMEM((2, CH, 128), jnp.float32),
            [pltpu.SemaphoreType.DMA] * 2,
        ],
        compiler_params=pltpu.CompilerParams(needs_layout_passes=False),
    )
    def gk(idxt_hbm, ltabt_hbm, out_hbm, idx_v, src_v, stg_v, wsems):
        wid = lax.axis_index("s") * nc + lax.axis_index("c")
        grp = wid // NWIN
        win = lax.rem(wid, NWIN)
        f0 = lax.min(grp * FG, F - FG)
        bb0 = win * 128

        pltpu.sync_copy(idxt_hbm.at[:, pl.ds(bb0, 128)], idx_v)

        def chunk_body(c, carry):
            fc = f0 + c * CH
            pltpu.sync_copy(ltabt_hbm.at[pl.ds(fc * Vpad, CH * Vpad)], src_v)

            def emit_l(l, par):
                t = c * L + l

                @pl.when(t >= 2)
                def _():
                    pltpu.make_async_copy(
                        stg_v.at[par],
                        out_hbm.at[0, pl.ds(f0, CH), pl.ds(bb0, 128)],
                        wsems[par],
                    ).wait()

                ivrow = idx_v.at[l]
                ivs = [ivrow[pl.ds(16 * j, 16)] for j in range(8)]

                def f_body(f, fcarry):
                    base = f * Vpad
                    srow = stg_v.at[par, f]
                    for j in range(8):
                        vals = plsc.load_gather(src_v, [ivs[j] + base])
                        srow[pl.ds(16 * j, 16)] = vals
                    return fcarry

                lax.fori_loop(0, CH, f_body, 0)
                pltpu.async_copy(
                    stg_v.at[par],
                    out_hbm.at[l, pl.ds(fc, CH), pl.ds(bb0, 128)],
                    wsems[par],
                )

            def l_body(i, lcarry):
                emit_l(2 * i, 0)
                emit_l(2 * i + 1, 1)
                return lcarry

            lax.fori_loop(0, L // 2, l_body, 0)
            return carry

        lax.fori_loop(0, NCH, chunk_body, 0)
        for par in range(2):
            pltpu.make_async_copy(
                stg_v.at[par],
                out_hbm.at[0, pl.ds(f0, CH), pl.ds(bb0, 128)],
                wsems[par],
            ).wait()

    return gk


def kernel(idx, table, W1, W2):
    B, L = idx.shape
    F = W2.shape[1]
    V = table.shape[0]
    Vpad = (V + 127) // 128 * 128
    ltabt = _compute_logits_t(table, W1, W2, Vpad)
    idxt = jnp.transpose(idx.astype(jnp.int32), (1, 0))
    out = _make_sc_tgather(L, F, B, Vpad)(idxt, ltabt.reshape(-1))
    return jnp.transpose(out, (2, 0, 1))


# speedup vs baseline: 2.6194x; 2.6194x over previous
import functools

import jax
import jax.numpy as jnp
from jax import lax
from jax.experimental import pallas as pl
from jax.experimental.pallas import tpu as pltpu
from jax.experimental.pallas import tpu_sc as plsc


def _mlp_t_body(tab_ref, w1_ref, w2_ref, out_ref):
    h = jnp.maximum(
        jnp.dot(tab_ref[...], w1_ref[...], preferred_element_type=jnp.float32),
        0.0,
    )
    t = lax.dot_general(
        w2_ref[...], h, (((0,), (1,)), ((), ())),
        preferred_element_type=jnp.float32,
    )
    out_ref[...] = jnp.concatenate(
        [t, jnp.zeros((t.shape[0], out_ref.shape[1] - t.shape[1]), jnp.float32)],
        axis=1,
    )


def _compute_logits_t(table, W1, W2, Vpad):
    V = table.shape[0]
    F = W2.shape[1]
    return pl.pallas_call(
        _mlp_t_body,
        out_shape=jax.ShapeDtypeStruct((F, Vpad), jnp.float32),
    )(table, W1, W2)


@functools.lru_cache(maxsize=None)
def _make_sc_tgather(L, F, BB, Vpad):
    info = plsc.get_sparse_core_info()
    nc, ns = info.num_cores, info.num_subcores
    nw = nc * ns
    NWIN = BB // 128
    NG = nw // NWIN
    CH = 64
    FG = 256
    NCH = FG // CH
    assert Vpad == 1024 and BB % 128 == 0 and NG * FG >= F and F >= FG

    mesh = plsc.VectorSubcoreMesh(core_axis_name="c", subcore_axis_name="s")

    @functools.partial(
        pl.kernel,
        mesh=mesh,
        out_type=jax.ShapeDtypeStruct((L, F, BB), jnp.float32),
        scratch_types=[
            pltpu.VMEM((L, 128), jnp.int32),
            pltpu.VMEM((CH * Vpad,), jnp.float32),
            pltpu.VMEM((2, CH, 128), jnp.float32),
            [pltpu.SemaphoreType.DMA] * 2,
        ],
        compiler_params=pltpu.CompilerParams(needs_layout_passes=False),
    )
    def gk(idxt_hbm, ltabt_hbm, out_hbm, idx_v, src_v, stg_v, wsems):
        wid = lax.axis_index("s") * nc + lax.axis_index("c")
        grp = wid // NWIN
        win = lax.rem(wid, NWIN)
        f0 = lax.min(grp * FG, F - FG)
        bb0 = win * 128

        pltpu.sync_copy(idxt_hbm.at[:, pl.ds(bb0, 128)], idx_v)

        def chunk_body(c, carry):
            fc = f0 + c * CH
            pltpu.sync_copy(ltabt_hbm.at[pl.ds(fc * Vpad, CH * Vpad)], src_v)

            def emit_l(l, par):
                t = c * L + l

                @pl.when(t >= 2)
                def _():
                    pltpu.make_async_copy(
                        stg_v.at[par],
                        out_hbm.at[0, pl.ds(f0, CH), pl.ds(bb0, 128)],
                        wsems[par],
                    ).wait()

                ivrow = idx_v.at[l]
                ivs = [ivrow[pl.ds(16 * j, 16)] for j in range(8)]

                def f_body(k, fcarry):
                    f = 4 * k
                    gs = []
                    for u in range(4):
                        base = (f + u) * Vpad
                        for j in range(8):
                            gs.append(plsc.load_gather(src_v, [ivs[j] + base]))
                    for u in range(4):
                        srow = stg_v.at[par, f + u]
                        for j in range(8):
                            srow[pl.ds(16 * j, 16)] = gs[8 * u + j]
                    return fcarry

                lax.fori_loop(0, CH // 4, f_body, 0)
                pltpu.async_copy(
                    stg_v.at[par],
                    out_hbm.at[l, pl.ds(fc, CH), pl.ds(bb0, 128)],
                    wsems[par],
                )

            def l_body(i, lcarry):
                emit_l(2 * i, 0)
                emit_l(2 * i + 1, 1)
                return lcarry

            lax.fori_loop(0, L // 2, l_body, 0)
            return carry

        lax.fori_loop(0, NCH, chunk_body, 0)
        for par in range(2):
            pltpu.make_async_copy(
                stg_v.at[par],
                out_hbm.at[0, pl.ds(f0, CH), pl.ds(bb0, 128)],
                wsems[par],
            ).wait()

    return gk


def kernel(idx, table, W1, W2):
    B, L = idx.shape
    F = W2.shape[1]
    V = table.shape[0]
    Vpad = (V + 127) // 128 * 128
    ltabt = _compute_logits_t(table, W1, W2, Vpad)
    idxt = jnp.transpose(idx.astype(jnp.int32), (1, 0))
    out = _make_sc_tgather(L, F, B, Vpad)(idxt, ltabt.reshape(-1))
    return jnp.transpose(out, (2, 0, 1))
